# baseline (speedup 1.0000x reference)
import functools
import jax
import jax.numpy as jnp
from jax import lax
from jax.experimental import pallas as pl
from jax.experimental.pallas import tpu as pltpu
from jax.experimental.pallas import tpu_sc as plsc

VOCAB = 100000
EMBED_DIM = 64
BATCH = 4096
SEQ = 200

PAD_DIM = 128
TOT = BATCH * SEQ
NW = 32
IDX_W = 128

N_SPLIT = 2
H = TOT // N_SPLIT
PER_W = H // NW
IDX_ROWS = PER_W // IDX_W
CHUNK = 640
K = CHUNK // IDX_W
NCH = PER_W // CHUNK
NBUF = 2

R_BLK = 4096
NB = H // R_BLK

_mesh = plsc.VectorSubcoreMesh(core_axis_name="c", subcore_axis_name="s")


@functools.partial(
    pl.kernel,
    mesh=_mesh,
    out_type=jax.ShapeDtypeStruct((H, PAD_DIM), jnp.float32),
    compiler_params=pltpu.CompilerParams(use_tc_tiling_on_sc=False),
    scratch_types=[
        pltpu.VMEM((IDX_ROWS, IDX_W), jnp.int32),
        pltpu.VMEM((NBUF, CHUNK, EMBED_DIM), jnp.float32),
        pltpu.SemaphoreType.DMA,
        pltpu.SemaphoreType.DMA,
        pltpu.SemaphoreType.DMA,
    ],
)
def _embed_gather(table_hbm, idx_hbm, out_hbm, idx_all, rows_v, sem_g,
                  sem_w0, sem_w1):
    wid = lax.axis_index("s") * 2 + lax.axis_index("c")
    row0 = wid * PER_W
    sem_w = (sem_w0, sem_w1)

    pltpu.sync_copy(idx_hbm.at[pl.ds(wid * IDX_ROWS, IDX_ROWS)], idx_all)

    def do_chunk(c, b):
        buf = rows_v.at[b]
        for j in range(K):
            pltpu.async_copy(
                table_hbm.at[idx_all.at[c * K + j]],
                buf.at[pl.ds(j * IDX_W, IDX_W)],
                sem_g,
            )
        for j in range(K):
            pltpu.make_async_copy(
                table_hbm.at[idx_all.at[c * K + j]],
                buf.at[pl.ds(j * IDX_W, IDX_W)],
                sem_g,
            ).wait()
        pltpu.async_copy(
            buf,
            out_hbm.at[pl.ds(row0 + c * CHUNK, CHUNK), pl.ds(0, EMBED_DIM)],
            sem_w[b])

    def wait_write(c, b):
        pltpu.make_async_copy(
            rows_v.at[b],
            out_hbm.at[pl.ds(row0 + c * CHUNK, CHUNK), pl.ds(0, EMBED_DIM)],
            sem_w[b],
        ).wait()

    for b in range(NBUF):
        do_chunk(b, b)

    @pl.loop(NBUF, NCH, step=NBUF)
    def _(cc):
        for b in range(NBUF):
            wait_write(cc + b - NBUF, b)
            do_chunk(cc + b, b)

    for b in range(NBUF):
        wait_write(NCH - NBUF + b, b)


def _compact_first(o_ref, out_ref):
    out_ref[...] = o_ref[:, :EMBED_DIM]


def _compact_chain(o_ref, prev_ref, out_ref):
    del prev_ref
    out_ref[...] = o_ref[:, :EMBED_DIM]


def _compact(piece, o, prev):
    base = piece * NB
    out_spec = pl.BlockSpec((R_BLK, EMBED_DIM), lambda i, base=base: (i + base, 0))
    in_spec = pl.BlockSpec((R_BLK, PAD_DIM), lambda i: (i, 0))
    if piece == 0:
        return pl.pallas_call(
            _compact_first,
            grid=(NB,),
            in_specs=[in_spec],
            out_specs=out_spec,
            out_shape=jax.ShapeDtypeStruct((TOT, EMBED_DIM), jnp.float32),
        )(o)
    return pl.pallas_call(
        _compact_chain,
        grid=(NB,),
        in_specs=[in_spec, pl.BlockSpec(memory_space=pl.ANY)],
        out_specs=out_spec,
        out_shape=jax.ShapeDtypeStruct((TOT, EMBED_DIM), jnp.float32),
        input_output_aliases={1: 0},
    )(o, prev)


def kernel(input, input_lengths, table):
    idx = jnp.asarray(input, jnp.int32).reshape(TOT // IDX_W, IDX_W)
    acc = None
    for p in range(N_SPLIT):
        rows = H // IDX_W
        o = _embed_gather(table, lax.slice(idx, (p * rows, 0),
                                           ((p + 1) * rows, IDX_W)))
        acc = _compact(p, o, acc)
    return (acc.reshape(BATCH, SEQ, EMBED_DIM), input_lengths)
